# baseline (speedup 1.0000x reference)
import jax
import jax.numpy as jnp
from jax import lax
from jax.experimental import pallas as pl
from jax.experimental.pallas import tpu as pltpu
from jax.experimental.pallas import tpu_sc as plsc

D = 64
L = 200
LP = 256
LANE = 16
NJ = D // LANE

_info = plsc.get_sparse_core_info()
NC = _info.num_cores
NS = _info.num_subcores
NW = NC * NS
CHUNK = 16


def _reduce_rows(rows_ref):
    def body(l, acc):
        return tuple(
            acc[j] + rows_ref[l, pl.ds(LANE * j, LANE)] for j in range(NJ))

    return lax.fori_loop(
        0, LP, body,
        tuple(jnp.zeros((LANE,), jnp.float32) for _ in range(NJ)))


def _pool_body(xf_hbm, emb_hbm, out_hbm, idx0, idx1, rows0, rows1, out_v,
               sem_i0, sem_i1, sem_g0, sem_g1):
    bpw = out_hbm.shape[0] // NW
    nchunk = bpw // CHUNK
    wid = lax.axis_index("s") * NC + lax.axis_index("c")
    base = wid * bpw
    maxid = base + bpw - 1

    idx = (idx0, idx1)
    rows = (rows0, rows1)
    sem_i = (sem_i0, sem_i1)
    sem_g = (sem_g0, sem_g1)

    def stage(gid, p):
        pltpu.async_copy(xf_hbm.at[pl.ds(gid * LP, LP)], idx[p], sem_i[p])

    def wait_stage(p):
        pltpu.make_async_copy(
            xf_hbm.at[pl.ds(0, LP)], idx[p], sem_i[p]).wait()

    def gather(p):
        pltpu.async_copy(emb_hbm.at[idx[p]], rows[p], sem_g[p])

    def wait_gather(p):
        pltpu.make_async_copy(emb_hbm.at[idx[p]], rows[p], sem_g[p]).wait()

    pltpu.sync_copy(xf_hbm.at[pl.ds(base * LP, LP)], idx0)
    gather(0)
    stage(jnp.minimum(base + 1, maxid), 1)

    def chunk_body(c, carry):
        cbase = base + c * CHUNK
        for e in range(CHUNK):
            pn = (e + 1) % 2
            pc = e % 2
            wait_stage(pn)
            gather(pn)
            wait_gather(pc)
            stage(jnp.minimum(cbase + e + 2, maxid), pc)
            acc = _reduce_rows(rows[pc])
            for j in range(NJ):
                out_v[e, pl.ds(LANE * j, LANE)] = acc[j]
        pltpu.sync_copy(out_v, out_hbm.at[pl.ds(cbase, CHUNK)])
        return carry

    lax.fori_loop(0, nchunk, chunk_body, 0)
    wait_gather(0)
    wait_stage(1)


def _pooled_sums(xf, emb, b):
    mesh = plsc.VectorSubcoreMesh(core_axis_name="c", subcore_axis_name="s")
    return pl.kernel(
        _pool_body,
        out_type=jax.ShapeDtypeStruct((b, D), jnp.float32),
        mesh=mesh,
        compiler_params=pltpu.CompilerParams(use_tc_tiling_on_sc=False),
        scratch_types=[
            pltpu.VMEM((LP,), jnp.int32),
            pltpu.VMEM((LP,), jnp.int32),
            pltpu.VMEM((LP, D), jnp.float32),
            pltpu.VMEM((LP, D), jnp.float32),
            pltpu.VMEM((CHUNK, D), jnp.float32),
            pltpu.SemaphoreType.DMA,
            pltpu.SemaphoreType.DMA,
            pltpu.SemaphoreType.DMA,
            pltpu.SemaphoreType.DMA,
        ],
    )(xf, emb)


def _mlp_body(s_ref, x_ref, e0_ref, w1_ref, b1_ref, w2_ref, b2_ref,
              w3t_ref, b3_ref, out_ref):
    count0 = jnp.sum((x_ref[...] == 0).astype(jnp.float32), axis=1,
                     keepdims=True)
    pooled = (s_ref[...] - count0 * e0_ref[...]) * jnp.float32(1.0 / L)
    h1 = jnp.maximum(
        jnp.dot(pooled, w1_ref[...], preferred_element_type=jnp.float32)
        + b1_ref[...], 0.0)
    h2 = jnp.maximum(
        jnp.dot(h1, w2_ref[...], preferred_element_type=jnp.float32)
        + b2_ref[...], 0.0)
    o = jnp.sum(h2 * w3t_ref[...], axis=1, keepdims=True) + b3_ref[...]
    out_ref[...] = jax.nn.sigmoid(o)


def _mlp(sums, xp, emb0, W1, b1, W2, b2, W3, b3):
    b = sums.shape[0]
    blk = 1024
    grid = b // blk
    h = W2.shape[1]
    out = pl.pallas_call(
        _mlp_body,
        grid=(grid,),
        in_specs=[
            pl.BlockSpec((blk, D), lambda i: (i, 0)),
            pl.BlockSpec((blk, LP), lambda i: (i, 0)),
            pl.BlockSpec((1, D), lambda i: (0, 0)),
            pl.BlockSpec((D, D), lambda i: (0, 0)),
            pl.BlockSpec((1, D), lambda i: (0, 0)),
            pl.BlockSpec((D, h), lambda i: (0, 0)),
            pl.BlockSpec((1, h), lambda i: (0, 0)),
            pl.BlockSpec((1, h), lambda i: (0, 0)),
            pl.BlockSpec((1, 1), lambda i: (0, 0)),
        ],
        out_specs=pl.BlockSpec((blk, 1), lambda i: (i, 0)),
        out_shape=jax.ShapeDtypeStruct((b, 1), jnp.float32),
    )(sums, xp, emb0, W1, b1.reshape(1, -1), W2, b2.reshape(1, -1),
      W3.reshape(1, -1), b3.reshape(1, 1))
    return out[:, 0]


def kernel(x, emb, W1, b1, W2, b2, W3, b3):
    xp = jnp.pad(x, ((0, 0), (0, LP - L)))
    sums = _pooled_sums(xp.reshape(-1), emb, x.shape[0])
    return _mlp(sums, xp, emb[0:1, :], W1, b1, W2, b2, W3, b3)


# speedup vs baseline: 18.0985x; 18.0985x over previous
import jax
import jax.numpy as jnp
from jax import lax
from jax.experimental import pallas as pl
from jax.experimental.pallas import tpu as pltpu
from jax.experimental.pallas import tpu_sc as plsc

D = 64
L = 200
LANE = 16
NJ = D // LANE
R = 8

_info = plsc.get_sparse_core_info()
NC = _info.num_cores
NS = _info.num_subcores
NW = NC * NS


def _reduce_rows(rows_ref):
    def body(l, acc):
        return tuple(
            acc[j] + rows_ref[l, pl.ds(LANE * j, LANE)] for j in range(NJ))

    return lax.fori_loop(
        0, L, body,
        tuple(jnp.zeros((LANE,), jnp.float32) for _ in range(NJ)))


def _pool_body(xf_hbm, emb_hbm, out_hbm, *scr):
    idx = scr[0:R]
    rows = scr[R:2 * R]
    out_v = scr[2 * R]
    sem_i = scr[2 * R + 1:3 * R + 1]
    sem_g = scr[3 * R + 1:4 * R + 1]

    bpw = out_hbm.shape[0] // NW
    niter = bpw // R
    wid = lax.axis_index("s") * NC + lax.axis_index("c")
    base = wid * bpw
    maxid = base + bpw - 1

    def stage(g, u):
        pltpu.async_copy(xf_hbm.at[pl.ds(g * L, L)], idx[u], sem_i[u])

    def wait_stage(u):
        pltpu.make_async_copy(
            xf_hbm.at[pl.ds(0, L)], idx[u], sem_i[u]).wait()

    def gather(u):
        pltpu.async_copy(emb_hbm.at[idx[u]], rows[u], sem_g[u])

    def wait_gather(u):
        pltpu.make_async_copy(emb_hbm.at[idx[u]], rows[u], sem_g[u]).wait()

    for u in range(R):
        stage(base + u, u)
    for u in range(R):
        wait_stage(u)
        gather(u)

    def body(t, carry):
        tbase = base + t * R
        for u in range(R):
            wait_gather(u)
            stage(jnp.minimum(tbase + R + u, maxid), u)
            acc = _reduce_rows(rows[u])
            for j in range(NJ):
                out_v[u, pl.ds(LANE * j, LANE)] = acc[j]
            wait_stage(u)
            gather(u)
        pltpu.sync_copy(out_v, out_hbm.at[pl.ds(tbase, R)])
        return carry

    lax.fori_loop(0, niter, body, 0)
    for u in range(R):
        wait_gather(u)


def _pooled_sums(xf, emb, b):
    mesh = plsc.VectorSubcoreMesh(core_axis_name="c", subcore_axis_name="s")
    scratch = (
        [pltpu.VMEM((L,), jnp.int32) for _ in range(R)]
        + [pltpu.VMEM((L, D), jnp.float32) for _ in range(R)]
        + [pltpu.VMEM((R, D), jnp.float32)]
        + [pltpu.SemaphoreType.DMA for _ in range(2 * R)]
    )
    return pl.kernel(
        _pool_body,
        out_type=jax.ShapeDtypeStruct((b, D), jnp.float32),
        mesh=mesh,
        compiler_params=pltpu.CompilerParams(use_tc_tiling_on_sc=False),
        scratch_types=scratch,
    )(xf, emb)


def _mlp_body(s_ref, x_ref, e0_ref, w1_ref, b1_ref, w2_ref, b2_ref,
              w3t_ref, b3_ref, out_ref):
    count0 = jnp.sum((x_ref[...] == 0).astype(jnp.float32), axis=1,
                     keepdims=True)
    pooled = (s_ref[...] - count0 * e0_ref[...]) * jnp.float32(1.0 / L)
    h1 = jnp.maximum(
        jnp.dot(pooled, w1_ref[...], preferred_element_type=jnp.float32)
        + b1_ref[...], 0.0)
    h2 = jnp.maximum(
        jnp.dot(h1, w2_ref[...], preferred_element_type=jnp.float32)
        + b2_ref[...], 0.0)
    o = jnp.sum(h2 * w3t_ref[...], axis=1, keepdims=True) + b3_ref[...]
    out_ref[...] = jax.nn.sigmoid(o)


def _mlp(sums, x, emb0, W1, b1, W2, b2, W3, b3):
    b = sums.shape[0]
    blk = 1024
    grid = b // blk
    h = W2.shape[1]
    out = pl.pallas_call(
        _mlp_body,
        grid=(grid,),
        in_specs=[
            pl.BlockSpec((blk, D), lambda i: (i, 0)),
            pl.BlockSpec((blk, L), lambda i: (i, 0)),
            pl.BlockSpec((1, D), lambda i: (0, 0)),
            pl.BlockSpec((D, D), lambda i: (0, 0)),
            pl.BlockSpec((1, D), lambda i: (0, 0)),
            pl.BlockSpec((D, h), lambda i: (0, 0)),
            pl.BlockSpec((1, h), lambda i: (0, 0)),
            pl.BlockSpec((1, h), lambda i: (0, 0)),
            pl.BlockSpec((1, 1), lambda i: (0, 0)),
        ],
        out_specs=pl.BlockSpec((blk, 1), lambda i: (i, 0)),
        out_shape=jax.ShapeDtypeStruct((b, 1), jnp.float32),
    )(sums, x, emb0, W1, b1.reshape(1, -1), W2, b2.reshape(1, -1),
      W3.reshape(1, -1), b3.reshape(1, 1))
    return out[:, 0]


def kernel(x, emb, W1, b1, W2, b2, W3, b3):
    sums = _pooled_sums(x.reshape(-1), emb, x.shape[0])
    return _mlp(sums, x, emb[0:1, :], W1, b1, W2, b2, W3, b3)


# speedup vs baseline: 18.9005x; 1.0443x over previous
import jax
import jax.numpy as jnp
from jax import lax
from jax.experimental import pallas as pl
from jax.experimental.pallas import tpu as pltpu
from jax.experimental.pallas import tpu_sc as plsc

D = 64
L = 200
LANE = 16
NJ = D // LANE
R = 8

_info = plsc.get_sparse_core_info()
NC = _info.num_cores
NS = _info.num_subcores
NW = NC * NS


def _reduce_rows(rows_ref):
    def body(l, acc):
        a = tuple(
            acc[j] + rows_ref[2 * l, pl.ds(LANE * j, LANE)]
            for j in range(NJ))
        b = tuple(
            acc[NJ + j] + rows_ref[2 * l + 1, pl.ds(LANE * j, LANE)]
            for j in range(NJ))
        return a + b

    acc = lax.fori_loop(
        0, L // 2, body,
        tuple(jnp.zeros((LANE,), jnp.float32) for _ in range(2 * NJ)))
    return tuple(acc[j] + acc[NJ + j] for j in range(NJ))


def _pool_body(xf_hbm, emb_hbm, out_hbm, *scr):
    idx = scr[0:R]
    rows = scr[R:2 * R]
    out_v = scr[2 * R]
    sem_i = scr[2 * R + 1:3 * R + 1]
    sem_g = scr[3 * R + 1:4 * R + 1]

    bpw = out_hbm.shape[0] // NW
    niter = bpw // R
    wid = lax.axis_index("s") * NC + lax.axis_index("c")
    base = wid * bpw
    maxid = base + bpw - 1

    def stage(g, u):
        pltpu.async_copy(xf_hbm.at[pl.ds(g * L, L)], idx[u], sem_i[u])

    def wait_stage(u):
        pltpu.make_async_copy(
            xf_hbm.at[pl.ds(0, L)], idx[u], sem_i[u]).wait()

    def gather(u):
        pltpu.async_copy(emb_hbm.at[idx[u]], rows[u], sem_g[u])

    def wait_gather(u):
        pltpu.make_async_copy(emb_hbm.at[idx[u]], rows[u], sem_g[u]).wait()

    for u in range(R):
        stage(base + u, u)
    for u in range(R):
        wait_stage(u)
        gather(u)

    def body(t, carry):
        tbase = base + t * R
        for u in range(R):
            wait_gather(u)
            stage(jnp.minimum(tbase + R + u, maxid), u)
            acc = _reduce_rows(rows[u])
            for j in range(NJ):
                out_v[u, pl.ds(LANE * j, LANE)] = acc[j]
            wait_stage(u)
            gather(u)
        pltpu.sync_copy(out_v, out_hbm.at[pl.ds(tbase, R)])
        return carry

    lax.fori_loop(0, niter, body, 0)
    for u in range(R):
        wait_gather(u)


def _pooled_sums(xf, emb, b):
    mesh = plsc.VectorSubcoreMesh(core_axis_name="c", subcore_axis_name="s")
    scratch = (
        [pltpu.VMEM((L,), jnp.int32) for _ in range(R)]
        + [pltpu.VMEM((L, D), jnp.float32) for _ in range(R)]
        + [pltpu.VMEM((R, D), jnp.float32)]
        + [pltpu.SemaphoreType.DMA for _ in range(2 * R)]
    )
    return pl.kernel(
        _pool_body,
        out_type=jax.ShapeDtypeStruct((b, D), jnp.float32),
        mesh=mesh,
        compiler_params=pltpu.CompilerParams(use_tc_tiling_on_sc=False),
        scratch_types=scratch,
    )(xf, emb)


def _mlp_body(s_ref, x_ref, e0_ref, w1_ref, b1_ref, w2_ref, b2_ref,
              w3t_ref, b3_ref, out_ref):
    count0 = jnp.sum((x_ref[...] == 0).astype(jnp.float32), axis=1,
                     keepdims=True)
    pooled = (s_ref[...] - count0 * e0_ref[...]) * jnp.float32(1.0 / L)
    h1 = jnp.maximum(
        jnp.dot(pooled, w1_ref[...], preferred_element_type=jnp.float32)
        + b1_ref[...], 0.0)
    h2 = jnp.maximum(
        jnp.dot(h1, w2_ref[...], preferred_element_type=jnp.float32)
        + b2_ref[...], 0.0)
    o = jnp.sum(h2 * w3t_ref[...], axis=1, keepdims=True) + b3_ref[...]
    out_ref[...] = jax.nn.sigmoid(o)


def _mlp(sums, x, emb0, W1, b1, W2, b2, W3, b3):
    b = sums.shape[0]
    blk = 1024
    grid = b // blk
    h = W2.shape[1]
    out = pl.pallas_call(
        _mlp_body,
        grid=(grid,),
        in_specs=[
            pl.BlockSpec((blk, D), lambda i: (i, 0)),
            pl.BlockSpec((blk, L), lambda i: (i, 0)),
            pl.BlockSpec((1, D), lambda i: (0, 0)),
            pl.BlockSpec((D, D), lambda i: (0, 0)),
            pl.BlockSpec((1, D), lambda i: (0, 0)),
            pl.BlockSpec((D, h), lambda i: (0, 0)),
            pl.BlockSpec((1, h), lambda i: (0, 0)),
            pl.BlockSpec((1, h), lambda i: (0, 0)),
            pl.BlockSpec((1, 1), lambda i: (0, 0)),
        ],
        out_specs=pl.BlockSpec((blk, 1), lambda i: (i, 0)),
        out_shape=jax.ShapeDtypeStruct((b, 1), jnp.float32),
    )(sums, x, emb0, W1, b1.reshape(1, -1), W2, b2.reshape(1, -1),
      W3.reshape(1, -1), b3.reshape(1, 1))
    return out[:, 0]


def kernel(x, emb, W1, b1, W2, b2, W3, b3):
    emb2 = lax.optimization_barrier(emb.reshape(-1)).reshape(emb.shape)
    sums = _pooled_sums(x.reshape(-1), emb2, x.shape[0])
    return _mlp(sums, x, emb2[0:1, :], W1, b1, W2, b2, W3, b3)
